# baseline (speedup 1.0000x reference)
import jax
import jax.numpy as jnp
from jax.experimental import pallas as pl


def _gcn_body(h_ref, a_ref, w_ref, b_ref, o_ref):
    a = a_ref[0]
    h = h_ref[0]
    w = w_ref[...]
    bias = b_ref[...]
    ab = a.astype(jnp.bfloat16)
    ones = jnp.ones((1, ab.shape[0]), dtype=jnp.bfloat16)
    deg = jnp.dot(ones, ab, preferred_element_type=jnp.float32)[0]
    dis = jnp.where(deg > 0, jax.lax.rsqrt(deg), 0.0)
    x = jnp.dot(h, w, preferred_element_type=jnp.float32)
    xs = (x * dis[:, None]).astype(jnp.bfloat16)
    z = jax.lax.dot_general(ab, xs, (((0,), (0,)), ((), ())),
                            preferred_element_type=jnp.float32)
    o_ref[0] = z * dis[:, None] + bias


def kernel(H, A, W, b):
    B, N, DIN = H.shape
    DOUT = W.shape[1]
    b2 = b.reshape(1, DOUT)
    return pl.pallas_call(
        _gcn_body,
        grid=(B,),
        in_specs=[
            pl.BlockSpec((1, N, DIN), lambda i: (i, 0, 0)),
            pl.BlockSpec((1, N, N), lambda i: (i, 0, 0)),
            pl.BlockSpec((DIN, DOUT), lambda i: (0, 0)),
            pl.BlockSpec((1, DOUT), lambda i: (0, 0)),
        ],
        out_specs=pl.BlockSpec((1, N, DOUT), lambda i: (i, 0, 0)),
        out_shape=jax.ShapeDtypeStruct((B, N, DOUT), jnp.float32),
    )(H, A, W, b2)


# speedup vs baseline: 1.1127x; 1.1127x over previous
import jax
import jax.numpy as jnp
from jax.experimental import pallas as pl


def _gcn_one(a, h, w, bias):
    deg = jnp.sum(a, axis=0)
    dis = jnp.where(deg > 0, jax.lax.rsqrt(deg), 0.0)
    x = jnp.dot(h, w, preferred_element_type=jnp.float32)
    xs = (x * dis[:, None]).astype(jnp.bfloat16)
    z = jax.lax.dot_general(a.astype(jnp.bfloat16), xs,
                            (((0,), (0,)), ((), ())),
                            preferred_element_type=jnp.float32)
    return z * dis[:, None] + bias


def _gcn_body(h_ref, a_ref, w_ref, b_ref, o_ref):
    w = w_ref[...]
    bias = b_ref[...]
    o_ref[0] = _gcn_one(a_ref[0], h_ref[0], w, bias)
    o_ref[1] = _gcn_one(a_ref[1], h_ref[1], w, bias)


def kernel(H, A, W, b):
    B, N, DIN = H.shape
    DOUT = W.shape[1]
    b2 = b.reshape(1, DOUT)
    return pl.pallas_call(
        _gcn_body,
        grid=(B // 2,),
        in_specs=[
            pl.BlockSpec((2, N, DIN), lambda i: (i, 0, 0)),
            pl.BlockSpec((2, N, N), lambda i: (i, 0, 0)),
            pl.BlockSpec((DIN, DOUT), lambda i: (0, 0)),
            pl.BlockSpec((1, DOUT), lambda i: (0, 0)),
        ],
        out_specs=pl.BlockSpec((2, N, DOUT), lambda i: (i, 0, 0)),
        out_shape=jax.ShapeDtypeStruct((B, N, DOUT), jnp.float32),
    )(H, A, W, b2)
